# Initial kernel scaffold; baseline (speedup 1.0000x reference)
import jax
import jax.numpy as jnp
from jax.experimental import pallas as pl


def kernel(x, edge_index, edge_type, w1, root1, b1, w2, root2, b2):
    raise NotImplementedError("write your pallas kernel here")


# speedup vs baseline: 7.1795x; 7.1795x over previous
import functools

import jax
import jax.numpy as jnp
from jax import lax
from jax.experimental import pallas as pl
from jax.experimental.pallas import tpu as pltpu
from jax.experimental.pallas import tpu_sc as plsc

NC = 2
NS = 16
CH = 32
B_EDGE = 80
IB = 25
ZR = 250
BN = 400


def _seg_body(with_cnt, R, N, G,
              xc0, xc1, xc2, xc3, src_hbm, comb_hbm, ones_hbm, zer_hbm, z8_hbm,
              *rest):
    if with_cnt:
        (S0, S1, S2, S3, cnt_hbm,
         src_v, comb_v, rows_v, ones_v, zbuf, z8buf, acc_sh, cnt_sh, sem) = rest
    else:
        (S0, S1, S2, S3,
         src_v, comb_v, rows_v, ones_v, zbuf, z8buf, acc_sh, cnt_sh, sem) = rest
        cnt_hbm = None

    core = lax.axis_index("c")
    tid = lax.axis_index("s")
    rn = R * N
    wt_tiles = 10
    wrows = rn // wt_tiles
    row0 = tid * wrows

    pltpu.sync_copy(ones_hbm, ones_v)
    pltpu.sync_copy(zer_hbm, zbuf)
    pltpu.sync_copy(z8_hbm, z8buf)

    xs = ((xc0, xc2), (xc1, xc3))
    Ss = ((S0, S2), (S1, S3))

    for p in range(2):
        @pl.when(tid < wt_tiles)
        def _():
            for j in range(wrows // ZR):
                pltpu.sync_copy(zbuf, acc_sh.at[pl.ds(row0 + j * ZR, ZR)])
            if with_cnt and p == 0:
                for j in range(wrows // ZR):
                    pltpu.sync_copy(z8buf, cnt_sh.at[pl.ds(row0 + j * ZR, ZR)])
        plsc.subcore_barrier()

        def edge_loop(x_hbm, count_too):
            def stage(st, carry):
                pltpu.sync_copy(src_hbm.at[tid, st], src_v)
                pltpu.sync_copy(comb_hbm.at[tid, st], comb_v)

                def step(g, c2):
                    pltpu.async_copy(x_hbm.at[src_v.at[g]], rows_v, sem).wait()
                    pltpu.sync_copy(rows_v, acc_sh.at[comb_v.at[g]], add=True)
                    if count_too:
                        pltpu.sync_copy(ones_v, cnt_sh.at[comb_v.at[g]],
                                        add=True)
                    return c2
                lax.fori_loop(0, IB, step, 0)
                return carry
            lax.fori_loop(0, G // IB, stage, 0)

        @pl.when(core == 0)
        def _():
            edge_loop(xs[p][0], with_cnt and p == 0)

        @pl.when(core == 1)
        def _():
            edge_loop(xs[p][1], False)

        plsc.subcore_barrier()

        @pl.when(jnp.logical_and(core == 0, tid < wt_tiles))
        def _():
            pltpu.sync_copy(acc_sh.at[pl.ds(row0, wrows)],
                            Ss[p][0].at[pl.ds(row0, wrows)])
            if with_cnt and p == 0:
                pltpu.sync_copy(cnt_sh.at[pl.ds(row0, wrows)],
                                cnt_hbm.at[pl.ds(row0, wrows)])

        @pl.when(jnp.logical_and(core == 1, tid < wt_tiles))
        def _():
            pltpu.sync_copy(acc_sh.at[pl.ds(row0, wrows)],
                            Ss[p][1].at[pl.ds(row0, wrows)])

        plsc.subcore_barrier()


def _make_seg(with_cnt, R, N, G):
    rn = R * N
    f32 = jnp.float32
    outs = [jax.ShapeDtypeStruct((rn, CH), f32) for _ in range(4)]
    if with_cnt:
        outs.append(jax.ShapeDtypeStruct((rn, 8), f32))
    scratch = [
        pltpu.VMEM((IB, B_EDGE), jnp.int32),
        pltpu.VMEM((IB, B_EDGE), jnp.int32),
        pltpu.VMEM((B_EDGE, CH), f32),
        pltpu.VMEM((B_EDGE, 8), f32),
        pltpu.VMEM((ZR, CH), f32),
        pltpu.VMEM((ZR, 8), f32),
        pltpu.VMEM_SHARED((rn, CH), f32),
        pltpu.VMEM_SHARED((rn, 8), f32),
        pltpu.SemaphoreType.DMA,
    ]
    mesh = plsc.VectorSubcoreMesh(core_axis_name="c", subcore_axis_name="s",
                                  num_cores=NC, num_subcores=NS)
    return pl.kernel(functools.partial(_seg_body, with_cnt, R, N, G),
                     out_type=outs, mesh=mesh, scratch_types=scratch,
                     compiler_params=pltpu.CompilerParams(
                         use_tc_tiling_on_sc=False))


def _dense_body(R, relu, emit_h, x_ref, s0, s1, s2, s3, cnt_ref,
                root_ref, w_ref, b_ref, out_ref, *h_refs):
    x = x_ref[...]
    acc = jnp.dot(x, root_ref[...], preferred_element_type=jnp.float32)
    mean = jnp.concatenate([s0[...], s1[...], s2[...], s3[...]], axis=-1)
    cnt = cnt_ref[...][:, :, 0]
    inv = 1.0 / jnp.maximum(cnt, 1.0)
    mean = mean * inv[:, :, None]
    for r in range(R):
        acc = acc + jnp.dot(mean[r], w_ref[r], preferred_element_type=jnp.float32)
    acc = acc + b_ref[...]
    if relu:
        acc = jnp.maximum(acc, 0.0)
    out_ref[...] = acc
    for c, h_ref in enumerate(h_refs):
        h_ref[...] = acc[:, c * CH:(c + 1) * CH]


def _make_dense(R, N, D, relu, emit_h):
    f32 = jnp.float32
    nblk = N // BN
    s_spec = pl.BlockSpec((R, BN, CH), lambda i: (0, i, 0))
    in_specs = [
        pl.BlockSpec((BN, D), lambda i: (i, 0)),
        s_spec, s_spec, s_spec, s_spec,
        pl.BlockSpec((R, BN, 8), lambda i: (0, i, 0)),
        pl.BlockSpec((D, D), lambda i: (0, 0)),
        pl.BlockSpec((R, D, D), lambda i: (0, 0, 0)),
        pl.BlockSpec((1, D), lambda i: (0, 0)),
    ]
    out_specs = [pl.BlockSpec((BN, D), lambda i: (i, 0))]
    out_shape = [jax.ShapeDtypeStruct((N, D), f32)]
    if emit_h:
        out_specs += [pl.BlockSpec((BN, CH), lambda i: (i, 0)) for _ in range(4)]
        out_shape += [jax.ShapeDtypeStruct((N, CH), f32) for _ in range(4)]
    return pl.pallas_call(
        functools.partial(_dense_body, R, relu, emit_h),
        grid=(nblk,),
        in_specs=in_specs,
        out_specs=out_specs,
        out_shape=out_shape,
        compiler_params=pltpu.CompilerParams(
            dimension_semantics=("parallel",)),
    )


@jax.jit
def kernel(x, edge_index, edge_type, w1, root1, b1, w2, root2, b2):
    N, D = x.shape
    R = w1.shape[0]
    E = edge_type.shape[0]
    tile_e = E // NS
    G = tile_e // B_EDGE
    f32 = jnp.float32

    src = edge_index[0].reshape(NS, G // IB, IB, B_EDGE)
    comb = (edge_type * N + edge_index[1]).reshape(NS, G // IB, IB, B_EDGE)
    ones8 = jnp.ones((B_EDGE, 8), f32)
    zer = jnp.zeros((ZR, CH), f32)
    z8 = jnp.zeros((ZR, 8), f32)
    xc = [x[:, c * CH:(c + 1) * CH] for c in range(4)]

    seg1 = _make_seg(True, R, N, G)(*xc, src, comb, ones8, zer, z8)
    S1 = [s.reshape(R, N, CH) for s in seg1[:4]]
    cnt = seg1[4].reshape(R, N, 8)

    d1 = _make_dense(R, N, D, True, True)(
        x, *S1, cnt, root1, w1, b1.reshape(1, D))
    h, hc = d1[0], d1[1:]

    seg2 = _make_seg(False, R, N, G)(*hc, src, comb, ones8, zer, z8)
    S2 = [s.reshape(R, N, CH) for s in seg2]

    out = _make_dense(R, N, D, False, False)(
        h, *S2, cnt, root2, w2, b2.reshape(1, D))
    return out[0]


# speedup vs baseline: 12.5381x; 1.7464x over previous
import functools

import jax
import jax.numpy as jnp
from jax import lax
from jax.experimental import pallas as pl
from jax.experimental.pallas import tpu as pltpu
from jax.experimental.pallas import tpu_sc as plsc

NC = 2
NS = 16
CH = 32
B_EDGE = 100
K_PIPE = 4
NGRP = 5
IB = NGRP * K_PIPE
ZR = 125
BN = 400


def _zero_acc(tid, wt_tiles, wrows, row0, zbuf, acc_sh):
    @pl.when(tid < wt_tiles)
    def _():
        for j in range(wrows // ZR):
            pltpu.sync_copy(zbuf, acc_sh.at[pl.ds(row0 + j * ZR, ZR)])


def _seg_body(with_cnt, R, N, G,
              xc0, xc1, xc2, xc3, src_hbm, comb_hbm, ones_hbm, zer_hbm,
              *rest):
    if with_cnt:
        (S0, S1, S2, S3, cnt0_hbm, cnt1_hbm,
         src_v, comb_v, rows_v, ones_v, zbuf, acc_sh, semg, sems) = rest
    else:
        (S0, S1, S2, S3,
         src_v, comb_v, rows_v, ones_v, zbuf, acc_sh, semg, sems) = rest
        cnt0_hbm = cnt1_hbm = None

    core = lax.axis_index("c")
    tid = lax.axis_index("s")
    rn = R * N
    nstg = G // IB
    wt_tiles = 10
    wrows = rn // wt_tiles
    row0 = tid * wrows

    pltpu.sync_copy(ones_hbm, ones_v)
    pltpu.sync_copy(zer_hbm, zbuf)

    xs = ((xc0, xc2), (xc1, xc3))
    Ss = ((S0, S2), (S1, S3))

    def edge_stage(x_hbm):
        pend = [None, None]
        for grp in range(NGRP):
            bs = (grp % 2) * K_PIPE
            if pend[grp % 2] is not None:
                for d in pend[grp % 2]:
                    d.wait()
            gds = [pltpu.async_copy(x_hbm.at[src_v.at[grp * K_PIPE + k]],
                                    rows_v.at[bs + k], semg)
                   for k in range(K_PIPE)]
            for d in gds:
                d.wait()
            pend[grp % 2] = [
                pltpu.async_copy(rows_v.at[bs + k],
                                 acc_sh.at[comb_v.at[grp * K_PIPE + k]],
                                 sems, add=True)
                for k in range(K_PIPE)]
        for sds in pend:
            if sds is not None:
                for d in sds:
                    d.wait()

    def edge_loop(x_hbm, st_lo, st_hi):
        def stage(st, carry):
            pltpu.sync_copy(src_hbm.at[tid, st], src_v)
            pltpu.sync_copy(comb_hbm.at[tid, st], comb_v)
            edge_stage(x_hbm)
            return carry
        lax.fori_loop(st_lo, st_hi, stage, 0)

    for p in range(2):
        _zero_acc(tid, wt_tiles, wrows, row0, zbuf, acc_sh)
        plsc.subcore_barrier()

        @pl.when(core == 0)
        def _():
            edge_loop(xs[p][0], 0, nstg)

        @pl.when(core == 1)
        def _():
            edge_loop(xs[p][1], 0, nstg)

        plsc.subcore_barrier()

        @pl.when(jnp.logical_and(core == 0, tid < wt_tiles))
        def _():
            pltpu.sync_copy(acc_sh.at[pl.ds(row0, wrows)],
                            Ss[p][0].at[pl.ds(row0, wrows)])

        @pl.when(jnp.logical_and(core == 1, tid < wt_tiles))
        def _():
            pltpu.sync_copy(acc_sh.at[pl.ds(row0, wrows)],
                            Ss[p][1].at[pl.ds(row0, wrows)])

        plsc.subcore_barrier()

    if with_cnt:
        _zero_acc(tid, wt_tiles, wrows, row0, zbuf, acc_sh)
        plsc.subcore_barrier()

        def cnt_stage(st, carry):
            pltpu.sync_copy(comb_hbm.at[tid, st], comb_v)
            sds = [pltpu.async_copy(ones_v,
                                    acc_sh.at[comb_v.at[g]], sems, add=True)
                   for g in range(IB)]
            for d in sds:
                d.wait()
            return carry

        half = nstg // 2
        lax.fori_loop(core * half, core * half + half, cnt_stage, 0)
        plsc.subcore_barrier()

        @pl.when(jnp.logical_and(core == 0, tid < wt_tiles))
        def _():
            pltpu.sync_copy(acc_sh.at[pl.ds(row0, wrows)],
                            cnt0_hbm.at[pl.ds(row0, wrows)])

        @pl.when(jnp.logical_and(core == 1, tid < wt_tiles))
        def _():
            pltpu.sync_copy(acc_sh.at[pl.ds(row0, wrows)],
                            cnt1_hbm.at[pl.ds(row0, wrows)])

        plsc.subcore_barrier()


def _make_seg(with_cnt, R, N, G):
    rn = R * N
    f32 = jnp.float32
    outs = [jax.ShapeDtypeStruct((rn, CH), f32) for _ in range(4)]
    if with_cnt:
        outs += [jax.ShapeDtypeStruct((rn, CH), f32) for _ in range(2)]
    scratch = [
        pltpu.VMEM((IB, B_EDGE), jnp.int32),
        pltpu.VMEM((IB, B_EDGE), jnp.int32),
        pltpu.VMEM((2 * K_PIPE, B_EDGE, CH), f32),
        pltpu.VMEM((B_EDGE, CH), f32),
        pltpu.VMEM((ZR, CH), f32),
        pltpu.VMEM_SHARED((rn, CH), f32),
        pltpu.SemaphoreType.DMA,
        pltpu.SemaphoreType.DMA,
    ]
    mesh = plsc.VectorSubcoreMesh(core_axis_name="c", subcore_axis_name="s",
                                  num_cores=NC, num_subcores=NS)
    return pl.kernel(functools.partial(_seg_body, with_cnt, R, N, G),
                     out_type=outs, mesh=mesh, scratch_types=scratch,
                     compiler_params=pltpu.CompilerParams(
                         use_tc_tiling_on_sc=False))


def _dense_body(R, relu, emit_h, x_ref, s0, s1, s2, s3, c0_ref, c1_ref,
                root_ref, w_ref, b_ref, out_ref, *h_refs):
    x = x_ref[...]
    acc = jnp.dot(x, root_ref[...], preferred_element_type=jnp.float32)
    mean = jnp.concatenate([s0[...], s1[...], s2[...], s3[...]], axis=-1)
    cnt = c0_ref[...][:, :, 0] + c1_ref[...][:, :, 0]
    inv = 1.0 / jnp.maximum(cnt, 1.0)
    mean = mean * inv[:, :, None]
    for r in range(R):
        acc = acc + jnp.dot(mean[r], w_ref[r], preferred_element_type=jnp.float32)
    acc = acc + b_ref[...]
    if relu:
        acc = jnp.maximum(acc, 0.0)
    out_ref[...] = acc
    for c, h_ref in enumerate(h_refs):
        h_ref[...] = acc[:, c * CH:(c + 1) * CH]


def _make_dense(R, N, D, relu, emit_h):
    f32 = jnp.float32
    nblk = N // BN
    s_spec = pl.BlockSpec((R, BN, CH), lambda i: (0, i, 0))
    in_specs = [
        pl.BlockSpec((BN, D), lambda i: (i, 0)),
        s_spec, s_spec, s_spec, s_spec,
        s_spec, s_spec,
        pl.BlockSpec((D, D), lambda i: (0, 0)),
        pl.BlockSpec((R, D, D), lambda i: (0, 0, 0)),
        pl.BlockSpec((1, D), lambda i: (0, 0)),
    ]
    out_specs = [pl.BlockSpec((BN, D), lambda i: (i, 0))]
    out_shape = [jax.ShapeDtypeStruct((N, D), f32)]
    if emit_h:
        out_specs += [pl.BlockSpec((BN, CH), lambda i: (i, 0)) for _ in range(4)]
        out_shape += [jax.ShapeDtypeStruct((N, CH), f32) for _ in range(4)]
    return pl.pallas_call(
        functools.partial(_dense_body, R, relu, emit_h),
        grid=(nblk,),
        in_specs=in_specs,
        out_specs=out_specs,
        out_shape=out_shape,
        compiler_params=pltpu.CompilerParams(
            dimension_semantics=("parallel",)),
    )


@jax.jit
def kernel(x, edge_index, edge_type, w1, root1, b1, w2, root2, b2):
    N, D = x.shape
    R = w1.shape[0]
    E = edge_type.shape[0]
    tile_e = E // NS
    G = tile_e // B_EDGE
    f32 = jnp.float32

    src = edge_index[0].reshape(NS, G // IB, IB, B_EDGE)
    comb = (edge_type * N + edge_index[1]).reshape(NS, G // IB, IB, B_EDGE)
    ones = jnp.ones((B_EDGE, CH), f32)
    zer = jnp.zeros((ZR, CH), f32)
    xc = [x[:, c * CH:(c + 1) * CH] for c in range(4)]

    seg1 = _make_seg(True, R, N, G)(*xc, src, comb, ones, zer)
    S1 = [s.reshape(R, N, CH) for s in seg1[:4]]
    cnt0 = seg1[4].reshape(R, N, CH)
    cnt1 = seg1[5].reshape(R, N, CH)

    d1 = _make_dense(R, N, D, True, True)(
        x, *S1, cnt0, cnt1, root1, w1, b1.reshape(1, D))
    h, hc = d1[0], d1[1:]

    seg2 = _make_seg(False, R, N, G)(*hc, src, comb, ones, zer)
    S2 = [s.reshape(R, N, CH) for s in seg2]

    out = _make_dense(R, N, D, False, False)(
        h, *S2, cnt0, cnt1, root2, w2, b2.reshape(1, D))
    return out[0]


# speedup vs baseline: 12.8988x; 1.0288x over previous
import functools

import jax
import jax.numpy as jnp
from jax import lax
from jax.experimental import pallas as pl
from jax.experimental.pallas import tpu as pltpu
from jax.experimental.pallas import tpu_sc as plsc

NC = 2
NS = 16
CH = 32
B_EDGE = 100
K_PIPE = 4
NGRP = 5
IB = NGRP * K_PIPE
ZR = 125
BN = 400


def _zero_acc(tid, wt_tiles, wrows, row0, zbuf, acc_sh):
    @pl.when(tid < wt_tiles)
    def _():
        for j in range(wrows // ZR):
            pltpu.sync_copy(zbuf, acc_sh.at[pl.ds(row0 + j * ZR, ZR)])


def _seg_body(with_cnt, R, N, G,
              xc0, xc1, xc2, xc3, src_hbm, comb_hbm, ones_hbm, zer_hbm,
              *rest):
    if with_cnt:
        (S0, S1, S2, S3, cnt0_hbm, cnt1_hbm,
         src_v, comb_v, rows_v, ones_v, zbuf, acc_sh,
         semg0, semg1, sems0, sems1) = rest
    else:
        (S0, S1, S2, S3,
         src_v, comb_v, rows_v, ones_v, zbuf, acc_sh,
         semg0, semg1, sems0, sems1) = rest
        cnt0_hbm = cnt1_hbm = None
    semg = (semg0, semg1)
    sems = (sems0, sems1)

    core = lax.axis_index("c")
    tid = lax.axis_index("s")
    rn = R * N
    nstg = G // IB
    wt_tiles = 10
    wrows = rn // wt_tiles
    row0 = tid * wrows

    pltpu.sync_copy(ones_hbm, ones_v)
    pltpu.sync_copy(zer_hbm, zbuf)

    xs = ((xc0, xc2), (xc1, xc3))
    Ss = ((S0, S2), (S1, S3))

    def edge_stage(x_hbm):
        pend = [None, None]

        def fire_g(grp):
            bs = (grp % 2) * K_PIPE
            return [pltpu.async_copy(x_hbm.at[src_v.at[grp * K_PIPE + k]],
                                     rows_v.at[bs + k], semg[grp % 2])
                    for k in range(K_PIPE)]

        gds = fire_g(0)
        for grp in range(NGRP):
            bs = (grp % 2) * K_PIPE
            nxt = None
            if grp + 1 < NGRP:
                if pend[(grp + 1) % 2] is not None:
                    for d in pend[(grp + 1) % 2]:
                        d.wait()
                    pend[(grp + 1) % 2] = None
                nxt = fire_g(grp + 1)
            for d in gds:
                d.wait()
            pend[grp % 2] = [
                pltpu.async_copy(rows_v.at[bs + k],
                                 acc_sh.at[comb_v.at[grp * K_PIPE + k]],
                                 sems[grp % 2], add=True)
                for k in range(K_PIPE)]
            gds = nxt
        for sds in pend:
            if sds is not None:
                for d in sds:
                    d.wait()

    def edge_loop(x_hbm, st_lo, st_hi):
        def stage(st, carry):
            pltpu.sync_copy(src_hbm.at[tid, st], src_v)
            pltpu.sync_copy(comb_hbm.at[tid, st], comb_v)
            edge_stage(x_hbm)
            return carry
        lax.fori_loop(st_lo, st_hi, stage, 0)

    for p in range(2):
        _zero_acc(tid, wt_tiles, wrows, row0, zbuf, acc_sh)
        plsc.subcore_barrier()

        @pl.when(core == 0)
        def _():
            edge_loop(xs[p][0], 0, nstg)

        @pl.when(core == 1)
        def _():
            edge_loop(xs[p][1], 0, nstg)

        plsc.subcore_barrier()

        @pl.when(jnp.logical_and(core == 0, tid < wt_tiles))
        def _():
            pltpu.sync_copy(acc_sh.at[pl.ds(row0, wrows)],
                            Ss[p][0].at[pl.ds(row0, wrows)])

        @pl.when(jnp.logical_and(core == 1, tid < wt_tiles))
        def _():
            pltpu.sync_copy(acc_sh.at[pl.ds(row0, wrows)],
                            Ss[p][1].at[pl.ds(row0, wrows)])

        plsc.subcore_barrier()

    if with_cnt:
        _zero_acc(tid, wt_tiles, wrows, row0, zbuf, acc_sh)
        plsc.subcore_barrier()

        def cnt_stage(st, carry):
            pltpu.sync_copy(comb_hbm.at[tid, st], comb_v)
            sds = [pltpu.async_copy(ones_v,
                                    acc_sh.at[comb_v.at[g]], sems[0], add=True)
                   for g in range(IB)]
            for d in sds:
                d.wait()
            return carry

        half = nstg // 2
        lax.fori_loop(core * half, core * half + half, cnt_stage, 0)
        plsc.subcore_barrier()

        @pl.when(jnp.logical_and(core == 0, tid < wt_tiles))
        def _():
            pltpu.sync_copy(acc_sh.at[pl.ds(row0, wrows)],
                            cnt0_hbm.at[pl.ds(row0, wrows)])

        @pl.when(jnp.logical_and(core == 1, tid < wt_tiles))
        def _():
            pltpu.sync_copy(acc_sh.at[pl.ds(row0, wrows)],
                            cnt1_hbm.at[pl.ds(row0, wrows)])

        plsc.subcore_barrier()


def _make_seg(with_cnt, R, N, G):
    rn = R * N
    f32 = jnp.float32
    outs = [jax.ShapeDtypeStruct((rn, CH), f32) for _ in range(4)]
    if with_cnt:
        outs += [jax.ShapeDtypeStruct((rn, CH), f32) for _ in range(2)]
    scratch = [
        pltpu.VMEM((IB, B_EDGE), jnp.int32),
        pltpu.VMEM((IB, B_EDGE), jnp.int32),
        pltpu.VMEM((2 * K_PIPE, B_EDGE, CH), f32),
        pltpu.VMEM((B_EDGE, CH), f32),
        pltpu.VMEM((ZR, CH), f32),
        pltpu.VMEM_SHARED((rn, CH), f32),
        pltpu.SemaphoreType.DMA,
        pltpu.SemaphoreType.DMA,
        pltpu.SemaphoreType.DMA,
        pltpu.SemaphoreType.DMA,
    ]
    mesh = plsc.VectorSubcoreMesh(core_axis_name="c", subcore_axis_name="s",
                                  num_cores=NC, num_subcores=NS)
    return pl.kernel(functools.partial(_seg_body, with_cnt, R, N, G),
                     out_type=outs, mesh=mesh, scratch_types=scratch,
                     compiler_params=pltpu.CompilerParams(
                         use_tc_tiling_on_sc=False))


def _dense_body(R, relu, emit_h, x_ref, s0, s1, s2, s3, c0_ref, c1_ref,
                root_ref, w_ref, b_ref, out_ref, *h_refs):
    x = x_ref[...]
    acc = jnp.dot(x, root_ref[...], preferred_element_type=jnp.float32)
    mean = jnp.concatenate([s0[...], s1[...], s2[...], s3[...]], axis=-1)
    cnt = c0_ref[...][:, :, 0] + c1_ref[...][:, :, 0]
    inv = 1.0 / jnp.maximum(cnt, 1.0)
    mean = mean * inv[:, :, None]
    for r in range(R):
        acc = acc + jnp.dot(mean[r], w_ref[r], preferred_element_type=jnp.float32)
    acc = acc + b_ref[...]
    if relu:
        acc = jnp.maximum(acc, 0.0)
    out_ref[...] = acc
    for c, h_ref in enumerate(h_refs):
        h_ref[...] = acc[:, c * CH:(c + 1) * CH]


def _make_dense(R, N, D, relu, emit_h):
    f32 = jnp.float32
    nblk = N // BN
    s_spec = pl.BlockSpec((R, BN, CH), lambda i: (0, i, 0))
    in_specs = [
        pl.BlockSpec((BN, D), lambda i: (i, 0)),
        s_spec, s_spec, s_spec, s_spec,
        s_spec, s_spec,
        pl.BlockSpec((D, D), lambda i: (0, 0)),
        pl.BlockSpec((R, D, D), lambda i: (0, 0, 0)),
        pl.BlockSpec((1, D), lambda i: (0, 0)),
    ]
    out_specs = [pl.BlockSpec((BN, D), lambda i: (i, 0))]
    out_shape = [jax.ShapeDtypeStruct((N, D), f32)]
    if emit_h:
        out_specs += [pl.BlockSpec((BN, CH), lambda i: (i, 0)) for _ in range(4)]
        out_shape += [jax.ShapeDtypeStruct((N, CH), f32) for _ in range(4)]
    return pl.pallas_call(
        functools.partial(_dense_body, R, relu, emit_h),
        grid=(nblk,),
        in_specs=in_specs,
        out_specs=out_specs,
        out_shape=out_shape,
        compiler_params=pltpu.CompilerParams(
            dimension_semantics=("parallel",)),
    )


@jax.jit
def kernel(x, edge_index, edge_type, w1, root1, b1, w2, root2, b2):
    N, D = x.shape
    R = w1.shape[0]
    E = edge_type.shape[0]
    tile_e = E // NS
    G = tile_e // B_EDGE
    f32 = jnp.float32

    src = edge_index[0].reshape(NS, G // IB, IB, B_EDGE)
    comb = (edge_type * N + edge_index[1]).reshape(NS, G // IB, IB, B_EDGE)
    ones = jnp.ones((B_EDGE, CH), f32)
    zer = jnp.zeros((ZR, CH), f32)
    xc = [x[:, c * CH:(c + 1) * CH] for c in range(4)]

    seg1 = _make_seg(True, R, N, G)(*xc, src, comb, ones, zer)
    S1 = [s.reshape(R, N, CH) for s in seg1[:4]]
    cnt0 = seg1[4].reshape(R, N, CH)
    cnt1 = seg1[5].reshape(R, N, CH)

    d1 = _make_dense(R, N, D, True, True)(
        x, *S1, cnt0, cnt1, root1, w1, b1.reshape(1, D))
    h, hc = d1[0], d1[1:]

    seg2 = _make_seg(False, R, N, G)(*hc, src, comb, ones, zer)
    S2 = [s.reshape(R, N, CH) for s in seg2]

    out = _make_dense(R, N, D, False, False)(
        h, *S2, cnt0, cnt1, root2, w2, b2.reshape(1, D))
    return out[0]


# speedup vs baseline: 13.7292x; 1.0644x over previous
import functools

import jax
import jax.numpy as jnp
from jax import lax
from jax.experimental import pallas as pl
from jax.experimental.pallas import tpu as pltpu
from jax.experimental.pallas import tpu_sc as plsc

NC = 2
NS = 16
CH = 32
B_EDGE = 100
K_PIPE = 4
NGRP = 5
IB = NGRP * K_PIPE
ZR = 125
BN = 1000


def _zero_acc(tid, wt_tiles, wrows, row0, zbuf, acc_sh):
    @pl.when(tid < wt_tiles)
    def _():
        for j in range(wrows // ZR):
            pltpu.sync_copy(zbuf, acc_sh.at[pl.ds(row0 + j * ZR, ZR)])


def _seg_body(with_cnt, R, N, G,
              xf, src0_hbm, src1_hbm, src2_hbm, src3_hbm, comb_hbm,
              ones_hbm, zer_hbm, *rest):
    if with_cnt:
        (S0, S1, S2, S3, cnt0_hbm, cnt1_hbm,
         src_v, comb_v, rows_v, ones_v, zbuf, acc_sh,
         semg0, semg1, sems0, sems1) = rest
    else:
        (S0, S1, S2, S3,
         src_v, comb_v, rows_v, ones_v, zbuf, acc_sh,
         semg0, semg1, sems0, sems1) = rest
        cnt0_hbm = cnt1_hbm = None
    semg = (semg0, semg1)
    sems = (sems0, sems1)

    core = lax.axis_index("c")
    tid = lax.axis_index("s")
    rn = R * N
    nstg = G // IB
    wt_tiles = 10
    wrows = rn // wt_tiles
    row0 = tid * wrows

    pltpu.sync_copy(ones_hbm, ones_v)
    pltpu.sync_copy(zer_hbm, zbuf)

    srcs = ((src0_hbm, src2_hbm), (src1_hbm, src3_hbm))
    Ss = ((S0, S2), (S1, S3))

    def edge_stage():
        pend = [None, None]

        def fire_g(grp):
            bs = (grp % 2) * K_PIPE
            return [pltpu.async_copy(xf.at[src_v.at[grp * K_PIPE + k]],
                                     rows_v.at[bs + k], semg[grp % 2])
                    for k in range(K_PIPE)]

        gds = fire_g(0)
        for grp in range(NGRP):
            bs = (grp % 2) * K_PIPE
            nxt = None
            if grp + 1 < NGRP:
                if pend[(grp + 1) % 2] is not None:
                    for d in pend[(grp + 1) % 2]:
                        d.wait()
                    pend[(grp + 1) % 2] = None
                nxt = fire_g(grp + 1)
            for d in gds:
                d.wait()
            pend[grp % 2] = [
                pltpu.async_copy(rows_v.at[bs + k],
                                 acc_sh.at[comb_v.at[grp * K_PIPE + k]],
                                 sems[grp % 2], add=True)
                for k in range(K_PIPE)]
            gds = nxt
        for sds in pend:
            if sds is not None:
                for d in sds:
                    d.wait()

    def edge_loop(src_hbm):
        def stage(st, carry):
            pltpu.sync_copy(src_hbm.at[tid, st], src_v)
            pltpu.sync_copy(comb_hbm.at[tid, st], comb_v)
            edge_stage()
            return carry
        lax.fori_loop(0, nstg, stage, 0)

    for p in range(2):
        _zero_acc(tid, wt_tiles, wrows, row0, zbuf, acc_sh)
        plsc.subcore_barrier()

        @pl.when(core == 0)
        def _():
            edge_loop(srcs[p][0])

        @pl.when(core == 1)
        def _():
            edge_loop(srcs[p][1])

        plsc.subcore_barrier()

        @pl.when(jnp.logical_and(core == 0, tid < wt_tiles))
        def _():
            pltpu.sync_copy(acc_sh.at[pl.ds(row0, wrows)],
                            Ss[p][0].at[pl.ds(row0, wrows)])

        @pl.when(jnp.logical_and(core == 1, tid < wt_tiles))
        def _():
            pltpu.sync_copy(acc_sh.at[pl.ds(row0, wrows)],
                            Ss[p][1].at[pl.ds(row0, wrows)])

        plsc.subcore_barrier()

    if with_cnt:
        _zero_acc(tid, wt_tiles, wrows, row0, zbuf, acc_sh)
        plsc.subcore_barrier()

        def cnt_stage(st, carry):
            pltpu.sync_copy(comb_hbm.at[tid, st], comb_v)
            sds = [pltpu.async_copy(ones_v,
                                    acc_sh.at[comb_v.at[g]], sems[0], add=True)
                   for g in range(IB)]
            for d in sds:
                d.wait()
            return carry

        half = nstg // 2
        lax.fori_loop(core * half, core * half + half, cnt_stage, 0)
        plsc.subcore_barrier()

        @pl.when(jnp.logical_and(core == 0, tid < wt_tiles))
        def _():
            pltpu.sync_copy(acc_sh.at[pl.ds(row0, wrows), pl.ds(0, 8)],
                            cnt0_hbm.at[pl.ds(row0, wrows)])

        @pl.when(jnp.logical_and(core == 1, tid < wt_tiles))
        def _():
            pltpu.sync_copy(acc_sh.at[pl.ds(row0, wrows), pl.ds(0, 8)],
                            cnt1_hbm.at[pl.ds(row0, wrows)])

        plsc.subcore_barrier()


def _make_seg(with_cnt, R, N, G):
    rn = R * N
    f32 = jnp.float32
    outs = [jax.ShapeDtypeStruct((rn, CH), f32) for _ in range(4)]
    if with_cnt:
        outs += [jax.ShapeDtypeStruct((rn, 8), f32) for _ in range(2)]
    scratch = [
        pltpu.VMEM((IB, B_EDGE), jnp.int32),
        pltpu.VMEM((IB, B_EDGE), jnp.int32),
        pltpu.VMEM((2 * K_PIPE, B_EDGE, CH), f32),
        pltpu.VMEM((B_EDGE, CH), f32),
        pltpu.VMEM((ZR, CH), f32),
        pltpu.VMEM_SHARED((rn, CH), f32),
        pltpu.SemaphoreType.DMA,
        pltpu.SemaphoreType.DMA,
        pltpu.SemaphoreType.DMA,
        pltpu.SemaphoreType.DMA,
    ]
    mesh = plsc.VectorSubcoreMesh(core_axis_name="c", subcore_axis_name="s",
                                  num_cores=NC, num_subcores=NS)
    return pl.kernel(functools.partial(_seg_body, with_cnt, R, N, G),
                     out_type=outs, mesh=mesh, scratch_types=scratch,
                     compiler_params=pltpu.CompilerParams(
                         use_tc_tiling_on_sc=False))


def _dense_body(R, relu, x_ref, s0, s1, s2, s3, c0_ref, c1_ref,
                root_ref, w_ref, b_ref, out_ref):
    x = x_ref[...]
    acc = jnp.dot(x, root_ref[...], preferred_element_type=jnp.float32)
    mean = jnp.concatenate([s0[...], s1[...], s2[...], s3[...]], axis=-1)
    cnt = c0_ref[...][:, :, 0] + c1_ref[...][:, :, 0]
    inv = 1.0 / jnp.maximum(cnt, 1.0)
    mean = mean * inv[:, :, None]
    for r in range(R):
        acc = acc + jnp.dot(mean[r], w_ref[r], preferred_element_type=jnp.float32)
    acc = acc + b_ref[...]
    if relu:
        acc = jnp.maximum(acc, 0.0)
    out_ref[...] = acc


def _make_dense(R, N, D, relu):
    f32 = jnp.float32
    nblk = N // BN
    s_spec = pl.BlockSpec((R, BN, CH), lambda i: (0, i, 0))
    in_specs = [
        pl.BlockSpec((BN, D), lambda i: (i, 0)),
        s_spec, s_spec, s_spec, s_spec,
        pl.BlockSpec((R, BN, 8), lambda i: (0, i, 0)),
        pl.BlockSpec((R, BN, 8), lambda i: (0, i, 0)),
        pl.BlockSpec((D, D), lambda i: (0, 0)),
        pl.BlockSpec((R, D, D), lambda i: (0, 0, 0)),
        pl.BlockSpec((1, D), lambda i: (0, 0)),
    ]
    return pl.pallas_call(
        functools.partial(_dense_body, R, relu),
        grid=(nblk,),
        in_specs=in_specs,
        out_specs=pl.BlockSpec((BN, D), lambda i: (i, 0)),
        out_shape=jax.ShapeDtypeStruct((N, D), f32),
        compiler_params=pltpu.CompilerParams(
            dimension_semantics=("parallel",)),
    )


@jax.jit
def kernel(x, edge_index, edge_type, w1, root1, b1, w2, root2, b2):
    N, D = x.shape
    R = w1.shape[0]
    E = edge_type.shape[0]
    tile_e = E // NS
    G = tile_e // B_EDGE
    f32 = jnp.float32

    src4 = edge_index[0] * 4
    srcs = [(src4 + c).reshape(NS, G // IB, IB, B_EDGE) for c in range(4)]
    comb = (edge_type * N + edge_index[1]).reshape(NS, G // IB, IB, B_EDGE)
    ones = jnp.ones((B_EDGE, CH), f32)
    zer = jnp.zeros((ZR, CH), f32)
    xf = x.reshape(4 * N, CH)

    seg1 = _make_seg(True, R, N, G)(xf, *srcs, comb, ones, zer)
    S1 = [s.reshape(R, N, CH) for s in seg1[:4]]
    cnt0 = seg1[4].reshape(R, N, 8)
    cnt1 = seg1[5].reshape(R, N, 8)

    h = _make_dense(R, N, D, True)(
        x, *S1, cnt0, cnt1, root1, w1, b1.reshape(1, D))

    seg2 = _make_seg(False, R, N, G)(h.reshape(4 * N, CH), *srcs, comb,
                                     ones, zer)
    S2 = [s.reshape(R, N, CH) for s in seg2]

    out = _make_dense(R, N, D, False)(
        h, *S2, cnt0, cnt1, root2, w2, b2.reshape(1, D))
    return out


# speedup vs baseline: 14.1890x; 1.0335x over previous
import functools

import jax
import jax.numpy as jnp
from jax import lax
from jax.experimental import pallas as pl
from jax.experimental.pallas import tpu as pltpu
from jax.experimental.pallas import tpu_sc as plsc

NC = 2
NS = 16
CH = 32
B_EDGE = 125
K_PIPE = 4
NGRP = 5
IB = NGRP * K_PIPE
ZR = 125
BN = 1000


def _zero_acc(tid, wt_tiles, wrows, row0, zbuf, acc_sh):
    @pl.when(tid < wt_tiles)
    def _():
        for j in range(wrows // ZR):
            pltpu.sync_copy(zbuf, acc_sh.at[pl.ds(row0 + j * ZR, ZR)])


def _seg_body(with_cnt, R, N, G,
              xf, src0_hbm, src1_hbm, src2_hbm, src3_hbm, comb_hbm,
              ones_hbm, zer_hbm, *rest):
    if with_cnt:
        (S0, S1, S2, S3, cnt0_hbm, cnt1_hbm,
         src_v, comb_v, rows_v, ones_v, zbuf, acc_sh,
         semg0, semg1, sems0, sems1) = rest
    else:
        (S0, S1, S2, S3,
         src_v, comb_v, rows_v, ones_v, zbuf, acc_sh,
         semg0, semg1, sems0, sems1) = rest
        cnt0_hbm = cnt1_hbm = None
    semg = (semg0, semg1)
    sems = (sems0, sems1)

    core = lax.axis_index("c")
    tid = lax.axis_index("s")
    rn = R * N
    nstg = G // IB
    wt_tiles = 10
    wrows = rn // wt_tiles
    row0 = tid * wrows

    pltpu.sync_copy(ones_hbm, ones_v)
    pltpu.sync_copy(zer_hbm, zbuf)

    srcs = ((src0_hbm, src2_hbm), (src1_hbm, src3_hbm))
    Ss = ((S0, S2), (S1, S3))

    def edge_stage():
        pend = [None, None]

        def fire_g(grp):
            bs = (grp % 2) * K_PIPE
            return [pltpu.async_copy(xf.at[src_v.at[grp * K_PIPE + k]],
                                     rows_v.at[bs + k], semg[grp % 2])
                    for k in range(K_PIPE)]

        gds = fire_g(0)
        for grp in range(NGRP):
            bs = (grp % 2) * K_PIPE
            nxt = None
            if grp + 1 < NGRP:
                if pend[(grp + 1) % 2] is not None:
                    for d in pend[(grp + 1) % 2]:
                        d.wait()
                    pend[(grp + 1) % 2] = None
                nxt = fire_g(grp + 1)
            for d in gds:
                d.wait()
            pend[grp % 2] = [
                pltpu.async_copy(rows_v.at[bs + k],
                                 acc_sh.at[comb_v.at[grp * K_PIPE + k]],
                                 sems[grp % 2], add=True)
                for k in range(K_PIPE)]
            gds = nxt
        for sds in pend:
            if sds is not None:
                for d in sds:
                    d.wait()

    def edge_loop(src_hbm):
        def stage(st, carry):
            pltpu.sync_copy(src_hbm.at[tid, st], src_v)
            pltpu.sync_copy(comb_hbm.at[tid, st], comb_v)
            edge_stage()
            return carry
        lax.fori_loop(0, nstg, stage, 0)

    for p in range(2):
        _zero_acc(tid, wt_tiles, wrows, row0, zbuf, acc_sh)
        plsc.subcore_barrier()

        @pl.when(core == 0)
        def _():
            edge_loop(srcs[p][0])

        @pl.when(core == 1)
        def _():
            edge_loop(srcs[p][1])

        plsc.subcore_barrier()

        @pl.when(jnp.logical_and(core == 0, tid < wt_tiles))
        def _():
            pltpu.sync_copy(acc_sh.at[pl.ds(row0, wrows)],
                            Ss[p][0].at[pl.ds(row0, wrows)])

        @pl.when(jnp.logical_and(core == 1, tid < wt_tiles))
        def _():
            pltpu.sync_copy(acc_sh.at[pl.ds(row0, wrows)],
                            Ss[p][1].at[pl.ds(row0, wrows)])

        plsc.subcore_barrier()

    if with_cnt:
        _zero_acc(tid, wt_tiles, wrows, row0, zbuf, acc_sh)
        plsc.subcore_barrier()

        def cnt_stage(st, carry):
            pltpu.sync_copy(comb_hbm.at[tid, st], comb_v)
            sds = [pltpu.async_copy(ones_v,
                                    acc_sh.at[comb_v.at[g]], sems[0], add=True)
                   for g in range(IB)]
            for d in sds:
                d.wait()
            return carry

        half = nstg // 2
        lax.fori_loop(core * half, core * half + half, cnt_stage, 0)
        plsc.subcore_barrier()

        @pl.when(jnp.logical_and(core == 0, tid < wt_tiles))
        def _():
            pltpu.sync_copy(acc_sh.at[pl.ds(row0, wrows), pl.ds(0, 8)],
                            cnt0_hbm.at[pl.ds(row0, wrows)])

        @pl.when(jnp.logical_and(core == 1, tid < wt_tiles))
        def _():
            pltpu.sync_copy(acc_sh.at[pl.ds(row0, wrows), pl.ds(0, 8)],
                            cnt1_hbm.at[pl.ds(row0, wrows)])

        plsc.subcore_barrier()


def _make_seg(with_cnt, R, N, G):
    rn = R * N
    f32 = jnp.float32
    outs = [jax.ShapeDtypeStruct((rn, CH), f32) for _ in range(4)]
    if with_cnt:
        outs += [jax.ShapeDtypeStruct((rn, 8), f32) for _ in range(2)]
    scratch = [
        pltpu.VMEM((IB, B_EDGE), jnp.int32),
        pltpu.VMEM((IB, B_EDGE), jnp.int32),
        pltpu.VMEM((2 * K_PIPE, B_EDGE, CH), f32),
        pltpu.VMEM((B_EDGE, CH), f32),
        pltpu.VMEM((ZR, CH), f32),
        pltpu.VMEM_SHARED((rn, CH), f32),
        pltpu.SemaphoreType.DMA,
        pltpu.SemaphoreType.DMA,
        pltpu.SemaphoreType.DMA,
        pltpu.SemaphoreType.DMA,
    ]
    mesh = plsc.VectorSubcoreMesh(core_axis_name="c", subcore_axis_name="s",
                                  num_cores=NC, num_subcores=NS)
    return pl.kernel(functools.partial(_seg_body, with_cnt, R, N, G),
                     out_type=outs, mesh=mesh, scratch_types=scratch,
                     compiler_params=pltpu.CompilerParams(
                         use_tc_tiling_on_sc=False))


def _dense_body(R, relu, x_ref, s0, s1, s2, s3, c0_ref, c1_ref,
                root_ref, w_ref, b_ref, out_ref):
    x = x_ref[...]
    acc = jnp.dot(x, root_ref[...], preferred_element_type=jnp.float32)
    mean = jnp.concatenate([s0[...], s1[...], s2[...], s3[...]], axis=-1)
    cnt = c0_ref[...][:, :, 0] + c1_ref[...][:, :, 0]
    inv = 1.0 / jnp.maximum(cnt, 1.0)
    mean = mean * inv[:, :, None]
    for r in range(R):
        acc = acc + jnp.dot(mean[r], w_ref[r], preferred_element_type=jnp.float32)
    acc = acc + b_ref[...]
    if relu:
        acc = jnp.maximum(acc, 0.0)
    out_ref[...] = acc


def _make_dense(R, N, D, relu):
    f32 = jnp.float32
    nblk = N // BN
    s_spec = pl.BlockSpec((R, BN, CH), lambda i: (0, i, 0))
    in_specs = [
        pl.BlockSpec((BN, D), lambda i: (i, 0)),
        s_spec, s_spec, s_spec, s_spec,
        pl.BlockSpec((R, BN, 8), lambda i: (0, i, 0)),
        pl.BlockSpec((R, BN, 8), lambda i: (0, i, 0)),
        pl.BlockSpec((D, D), lambda i: (0, 0)),
        pl.BlockSpec((R, D, D), lambda i: (0, 0, 0)),
        pl.BlockSpec((1, D), lambda i: (0, 0)),
    ]
    return pl.pallas_call(
        functools.partial(_dense_body, R, relu),
        grid=(nblk,),
        in_specs=in_specs,
        out_specs=pl.BlockSpec((BN, D), lambda i: (i, 0)),
        out_shape=jax.ShapeDtypeStruct((N, D), f32),
        compiler_params=pltpu.CompilerParams(
            dimension_semantics=("parallel",)),
    )


@jax.jit
def kernel(x, edge_index, edge_type, w1, root1, b1, w2, root2, b2):
    N, D = x.shape
    R = w1.shape[0]
    E = edge_type.shape[0]
    tile_e = E // NS
    G = tile_e // B_EDGE
    f32 = jnp.float32

    src4 = edge_index[0] * 4
    srcs = [(src4 + c).reshape(NS, G // IB, IB, B_EDGE) for c in range(4)]
    comb = (edge_type * N + edge_index[1]).reshape(NS, G // IB, IB, B_EDGE)
    ones = jnp.ones((B_EDGE, CH), f32)
    zer = jnp.zeros((ZR, CH), f32)
    xf = x.reshape(4 * N, CH)

    seg1 = _make_seg(True, R, N, G)(xf, *srcs, comb, ones, zer)
    S1 = [s.reshape(R, N, CH) for s in seg1[:4]]
    cnt0 = seg1[4].reshape(R, N, 8)
    cnt1 = seg1[5].reshape(R, N, 8)

    h = _make_dense(R, N, D, True)(
        x, *S1, cnt0, cnt1, root1, w1, b1.reshape(1, D))

    seg2 = _make_seg(False, R, N, G)(h.reshape(4 * N, CH), *srcs, comb,
                                     ones, zer)
    S2 = [s.reshape(R, N, CH) for s in seg2]

    out = _make_dense(R, N, D, False)(
        h, *S2, cnt0, cnt1, root2, w2, b2.reshape(1, D))
    return out
